# Initial kernel scaffold; baseline (speedup 1.0000x reference)
import jax
import jax.numpy as jnp
from jax.experimental import pallas as pl


def kernel(seg_pred, seg_targets):
    raise NotImplementedError("write your pallas kernel here")


# speedup vs baseline: 38.5093x; 38.5093x over previous
import functools
import math

import jax
import jax.numpy as jnp
from jax import lax
from jax.experimental import pallas as pl
from jax.experimental.pallas import tpu as pltpu
from jax.experimental.pallas import tpu_sc as plsc

_B = 16
_C = 19
_H = 512
_W = 512
_N = _B * _H * _W
_K = 100000
_C0 = float(-math.log(0.7))

_BH = 64


def _ce_body(write_loss, pred_ref, tgt_ref, *out_refs):
    if write_loss:
        loss_ref, sum_ref, cnt_ref = out_refs
    else:
        sum_ref, cnt_ref = out_refs
    i = pl.program_id(0)
    j = pl.program_id(1)
    x = pred_ref[0]
    t = tgt_ref[0]
    m = jnp.max(x, axis=0)
    e = jnp.exp(x - m[None, :, :])
    s = jnp.sum(e, axis=0)
    cc = lax.broadcasted_iota(jnp.int32, x.shape, 0)
    xt = jnp.sum(jnp.where(cc == t[None, :, :], x, 0.0), axis=0)
    l = jnp.log(s) + (m - xt)
    if write_loss:
        loss_ref[0] = l
    keep = l > _C0
    psum = jnp.sum(jnp.where(keep, l, 0.0))
    pcnt = jnp.sum(keep.astype(jnp.float32))

    @pl.when(jnp.logical_and(i == 0, j == 0))
    def _():
        sum_ref[0, 0] = 0.0
        cnt_ref[0, 0] = 0.0

    sum_ref[0, 0] += psum
    cnt_ref[0, 0] += pcnt


def _make_stage1(write_loss):
    out_shape = [
        jax.ShapeDtypeStruct((1, 1), jnp.float32),
        jax.ShapeDtypeStruct((1, 1), jnp.float32),
    ]
    out_specs = [
        pl.BlockSpec((1, 1), lambda i, j: (0, 0), memory_space=pltpu.SMEM),
        pl.BlockSpec((1, 1), lambda i, j: (0, 0), memory_space=pltpu.SMEM),
    ]
    if write_loss:
        out_shape = [jax.ShapeDtypeStruct((_B, _H, _W), jnp.float32)] + out_shape
        out_specs = [pl.BlockSpec((1, _BH, _W), lambda i, j: (i, j, 0))] + out_specs
    return pl.pallas_call(
        functools.partial(_ce_body, write_loss),
        grid=(_B, _H // _BH),
        in_specs=[
            pl.BlockSpec((1, _C, _BH, _W), lambda i, j: (i, 0, j, 0)),
            pl.BlockSpec((1, _BH, _W), lambda i, j: (i, j, 0)),
        ],
        out_specs=out_specs,
        out_shape=out_shape,
        compiler_params=pltpu.CompilerParams(
            dimension_semantics=("arbitrary", "arbitrary"),
        ),
    )


_NTILES = 32
_PER_TILE = _N // _NTILES
_CHUNK = 8192
_NBINS = 2048


def _make_hist(level):
    mesh = plsc.VectorSubcoreMesh(core_axis_name="c", subcore_axis_name="s")

    @functools.partial(
        pl.kernel,
        mesh=mesh,
        out_type=(
            jax.ShapeDtypeStruct((_NTILES, _NBINS), jnp.int32),
            jax.ShapeDtypeStruct((_NTILES, _NBINS), jnp.float32),
        ),
        scratch_types=(
            pltpu.VMEM((_CHUNK,), jnp.float32),
            pltpu.VMEM((_NBINS,), jnp.int32),
            pltpu.VMEM((_NBINS,), jnp.float32),
            pltpu.VMEM((16,), jnp.int32),
        ),
        compiler_params=pltpu.CompilerParams(needs_layout_passes=False),
    )
    def hist_k(loss_hbm, pref_hbm, cnt_hbm, sum_hbm, buf, hcnt, hsum, prefv):
        cid = lax.axis_index("c")
        sid = lax.axis_index("s")
        wid = sid * 2 + cid
        base = wid * _PER_TILE
        pltpu.sync_copy(pref_hbm, prefv)
        pv = prefv[...]

        def zero_body(i, carry):
            hcnt[pl.ds(i * 16, 16)] = jnp.zeros((16,), jnp.int32)
            hsum[pl.ds(i * 16, 16)] = jnp.zeros((16,), jnp.float32)
            return carry

        lax.fori_loop(0, _NBINS // 16, zero_body, 0)

        def chunk_body(cix, carry):
            pltpu.sync_copy(loss_hbm.at[pl.ds(base + cix * _CHUNK, _CHUNK)], buf)

            def grp(i, carry2):
                v = buf[pl.ds(i * 16, 16)]
                kb = lax.bitcast_convert_type(v, jnp.int32)
                key = kb ^ (
                    lax.shift_right_arithmetic(kb, 31) & jnp.int32(0x7FFFFFFF)
                )
                if level == 0:
                    mask = None
                    binv = lax.shift_right_arithmetic(key, 21) + 1024
                elif level == 1:
                    mask = lax.shift_right_arithmetic(key, 21) == pv
                    binv = lax.shift_right_logical(key, 10) & jnp.int32(2047)
                else:
                    mask = lax.shift_right_arithmetic(key, 10) == pv
                    binv = key & jnp.int32(1023)
                plsc.addupdate_scatter(
                    hcnt, [binv], jnp.ones((16,), jnp.int32), mask=mask
                )
                plsc.addupdate_scatter(hsum, [binv], v, mask=mask)
                return carry2

            lax.fori_loop(0, _CHUNK // 16, grp, 0)
            return carry

        lax.fori_loop(0, _PER_TILE // _CHUNK, chunk_body, 0)
        pltpu.sync_copy(hcnt, cnt_hbm.at[wid])
        pltpu.sync_copy(hsum, sum_hbm.at[wid])

    return hist_k


@functools.lru_cache(maxsize=None)
def _hist_kernel(level):
    return _make_hist(level)


def _level_reduce(cnt_rows, sum_rows, k):
    cnt = jnp.sum(cnt_rows, axis=0)
    sm = jnp.sum(sum_rows, axis=0)
    cge = jnp.cumsum(cnt[::-1])[::-1]
    sge = jnp.cumsum(sm[::-1])[::-1]
    meets = cge >= (k + 1)
    b = jnp.sum(meets.astype(jnp.int32)) - 1
    above_cnt = cge[b] - cnt[b]
    above_sum = sge[b] - sm[b]
    k_next = k - above_cnt
    return b, k_next, above_cnt, above_sum


def _select_cold(loss_flat):
    z16 = jnp.zeros((16,), jnp.int32)
    c1, s1 = _hist_kernel(0)(loss_flat, z16)
    b1, k1, ac1, as1 = _level_reduce(c1, s1, jnp.int32(_K))
    p1val = b1 - 1024
    c2, s2 = _hist_kernel(1)(loss_flat, jnp.full((16,), 0, jnp.int32) + p1val)
    b2, k2, ac2, as2 = _level_reduce(c2, s2, k1)
    p2val = (p1val << 11) | b2
    c3, s3 = _hist_kernel(2)(loss_flat, jnp.full((16,), 0, jnp.int32) + p2val)
    b3, k3, ac3, as3 = _level_reduce(c3, s3, k2)
    kept_cnt = ac1 + ac2 + ac3
    kept_sum = as1 + as2 + as3
    return kept_sum / jnp.maximum(kept_cnt, 1).astype(jnp.float32)


def kernel(seg_pred, seg_targets):
    psum, pcnt = _make_stage1(False)(seg_pred, seg_targets)
    total_sum = psum[0, 0]
    total_cnt = pcnt[0, 0]

    def _hot(_):
        return total_sum / total_cnt

    def _cold(_):
        outs = _make_stage1(True)(seg_pred, seg_targets)
        loss_flat = outs[0].reshape(-1)
        return _select_cold(loss_flat)

    return lax.cond(total_cnt > float(_K), _hot, _cold, None)


# speedup vs baseline: 42.7296x; 1.1096x over previous
import functools
import math

import jax
import jax.numpy as jnp
from jax import lax
from jax.experimental import pallas as pl
from jax.experimental.pallas import tpu as pltpu
from jax.experimental.pallas import tpu_sc as plsc

_B = 16
_C = 19
_H = 512
_W = 512
_N = _B * _H * _W
_K = 100000
_C0 = float(-math.log(0.7))

_BH = 64


def _ce_body(write_loss, pred_ref, tgt_ref, *out_refs):
    if write_loss:
        loss_ref, sum_ref, cnt_ref = out_refs
    else:
        sum_ref, cnt_ref = out_refs
    i = pl.program_id(0)
    j = pl.program_id(1)
    sub = 8
    acc_s = jnp.zeros((sub, _W), jnp.float32)
    acc_c = jnp.zeros((sub, _W), jnp.float32)
    for r in range(_BH // sub):
        rs = pl.ds(r * sub, sub)
        t = tgt_ref[0, rs]
        m = pred_ref[0, 0, rs]
        for c in range(1, _C):
            m = jnp.maximum(m, pred_ref[0, c, rs])
        s = jnp.zeros_like(m)
        xt = jnp.zeros_like(m)
        for c in range(_C):
            xc = pred_ref[0, c, rs]
            s = s + jnp.exp(xc - m)
            xt = jnp.where(t == c, xc, xt)
        l = jnp.log(s) + (m - xt)
        if write_loss:
            loss_ref[0, rs] = l
        keep = l > _C0
        acc_s = acc_s + jnp.where(keep, l, 0.0)
        acc_c = acc_c + keep.astype(jnp.float32)
    psum = jnp.sum(acc_s)
    pcnt = jnp.sum(acc_c)

    @pl.when(jnp.logical_and(i == 0, j == 0))
    def _():
        sum_ref[0, 0] = 0.0
        cnt_ref[0, 0] = 0.0

    sum_ref[0, 0] += psum
    cnt_ref[0, 0] += pcnt


def _make_stage1(write_loss):
    out_shape = [
        jax.ShapeDtypeStruct((1, 1), jnp.float32),
        jax.ShapeDtypeStruct((1, 1), jnp.float32),
    ]
    out_specs = [
        pl.BlockSpec((1, 1), lambda i, j: (0, 0), memory_space=pltpu.SMEM),
        pl.BlockSpec((1, 1), lambda i, j: (0, 0), memory_space=pltpu.SMEM),
    ]
    if write_loss:
        out_shape = [jax.ShapeDtypeStruct((_B, _H, _W), jnp.float32)] + out_shape
        out_specs = [pl.BlockSpec((1, _BH, _W), lambda i, j: (i, j, 0))] + out_specs
    return pl.pallas_call(
        functools.partial(_ce_body, write_loss),
        grid=(_B, _H // _BH),
        in_specs=[
            pl.BlockSpec((1, _C, _BH, _W), lambda i, j: (i, 0, j, 0)),
            pl.BlockSpec((1, _BH, _W), lambda i, j: (i, j, 0)),
        ],
        out_specs=out_specs,
        out_shape=out_shape,
        compiler_params=pltpu.CompilerParams(
            dimension_semantics=("arbitrary", "arbitrary"),
        ),
    )


_NTILES = 32
_PER_TILE = _N // _NTILES
_CHUNK = 8192
_NBINS = 2048


def _make_hist(level):
    mesh = plsc.VectorSubcoreMesh(core_axis_name="c", subcore_axis_name="s")

    @functools.partial(
        pl.kernel,
        mesh=mesh,
        out_type=(
            jax.ShapeDtypeStruct((_NTILES, _NBINS), jnp.int32),
            jax.ShapeDtypeStruct((_NTILES, _NBINS), jnp.float32),
        ),
        scratch_types=(
            pltpu.VMEM((_CHUNK,), jnp.float32),
            pltpu.VMEM((_NBINS,), jnp.int32),
            pltpu.VMEM((_NBINS,), jnp.float32),
            pltpu.VMEM((16,), jnp.int32),
        ),
        compiler_params=pltpu.CompilerParams(needs_layout_passes=False),
    )
    def hist_k(loss_hbm, pref_hbm, cnt_hbm, sum_hbm, buf, hcnt, hsum, prefv):
        cid = lax.axis_index("c")
        sid = lax.axis_index("s")
        wid = sid * 2 + cid
        base = wid * _PER_TILE
        pltpu.sync_copy(pref_hbm, prefv)
        pv = prefv[...]

        def zero_body(i, carry):
            hcnt[pl.ds(i * 16, 16)] = jnp.zeros((16,), jnp.int32)
            hsum[pl.ds(i * 16, 16)] = jnp.zeros((16,), jnp.float32)
            return carry

        lax.fori_loop(0, _NBINS // 16, zero_body, 0)

        def chunk_body(cix, carry):
            pltpu.sync_copy(loss_hbm.at[pl.ds(base + cix * _CHUNK, _CHUNK)], buf)

            def grp(i, carry2):
                v = buf[pl.ds(i * 16, 16)]
                kb = lax.bitcast_convert_type(v, jnp.int32)
                key = kb ^ (
                    lax.shift_right_arithmetic(kb, 31) & jnp.int32(0x7FFFFFFF)
                )
                if level == 0:
                    mask = None
                    binv = lax.shift_right_arithmetic(key, 21) + 1024
                elif level == 1:
                    mask = lax.shift_right_arithmetic(key, 21) == pv
                    binv = lax.shift_right_logical(key, 10) & jnp.int32(2047)
                else:
                    mask = lax.shift_right_arithmetic(key, 10) == pv
                    binv = key & jnp.int32(1023)
                plsc.addupdate_scatter(
                    hcnt, [binv], jnp.ones((16,), jnp.int32), mask=mask
                )
                plsc.addupdate_scatter(hsum, [binv], v, mask=mask)
                return carry2

            lax.fori_loop(0, _CHUNK // 16, grp, 0)
            return carry

        lax.fori_loop(0, _PER_TILE // _CHUNK, chunk_body, 0)
        pltpu.sync_copy(hcnt, cnt_hbm.at[wid])
        pltpu.sync_copy(hsum, sum_hbm.at[wid])

    return hist_k


@functools.lru_cache(maxsize=None)
def _hist_kernel(level):
    return _make_hist(level)


def _level_reduce(cnt_rows, sum_rows, k):
    cnt = jnp.sum(cnt_rows, axis=0)
    sm = jnp.sum(sum_rows, axis=0)
    cge = jnp.cumsum(cnt[::-1])[::-1]
    sge = jnp.cumsum(sm[::-1])[::-1]
    meets = cge >= (k + 1)
    b = jnp.sum(meets.astype(jnp.int32)) - 1
    above_cnt = cge[b] - cnt[b]
    above_sum = sge[b] - sm[b]
    k_next = k - above_cnt
    return b, k_next, above_cnt, above_sum


def _select_cold(loss_flat):
    z16 = jnp.zeros((16,), jnp.int32)
    c1, s1 = _hist_kernel(0)(loss_flat, z16)
    b1, k1, ac1, as1 = _level_reduce(c1, s1, jnp.int32(_K))
    p1val = b1 - 1024
    c2, s2 = _hist_kernel(1)(loss_flat, jnp.full((16,), 0, jnp.int32) + p1val)
    b2, k2, ac2, as2 = _level_reduce(c2, s2, k1)
    p2val = (p1val << 11) | b2
    c3, s3 = _hist_kernel(2)(loss_flat, jnp.full((16,), 0, jnp.int32) + p2val)
    b3, k3, ac3, as3 = _level_reduce(c3, s3, k2)
    kept_cnt = ac1 + ac2 + ac3
    kept_sum = as1 + as2 + as3
    return kept_sum / jnp.maximum(kept_cnt, 1).astype(jnp.float32)


def kernel(seg_pred, seg_targets):
    psum, pcnt = _make_stage1(False)(seg_pred, seg_targets)
    total_sum = psum[0, 0]
    total_cnt = pcnt[0, 0]

    def _hot(_):
        return total_sum / total_cnt

    def _cold(_):
        outs = _make_stage1(True)(seg_pred, seg_targets)
        loss_flat = outs[0].reshape(-1)
        return _select_cold(loss_flat)

    return lax.cond(total_cnt > float(_K), _hot, _cold, None)


# speedup vs baseline: 54.0104x; 1.2640x over previous
import functools
import math

import jax
import jax.numpy as jnp
from jax import lax
from jax.experimental import pallas as pl
from jax.experimental.pallas import tpu as pltpu
from jax.experimental.pallas import tpu_sc as plsc

_B = 16
_C = 19
_H = 512
_W = 512
_N = _B * _H * _W
_K = 100000
_C0 = float(-math.log(0.7))

_BH = 128


def _ce_body(write_loss, pred_ref, tgt_ref, *out_refs):
    if write_loss:
        loss_ref, sum_ref, cnt_ref = out_refs
    else:
        sum_ref, cnt_ref = out_refs
    i = pl.program_id(0)
    j = pl.program_id(1)
    sub = 8
    acc_s = jnp.zeros((sub, _W), jnp.float32)
    acc_c = jnp.zeros((sub, _W), jnp.float32)
    for r in range(_BH // sub):
        rs = pl.ds(r * sub, sub)
        t = tgt_ref[0, rs]
        m = pred_ref[0, 0, rs]
        for c in range(1, _C):
            m = jnp.maximum(m, pred_ref[0, c, rs])
        s = jnp.zeros_like(m)
        xt = jnp.zeros_like(m)
        for c in range(_C):
            xc = pred_ref[0, c, rs]
            s = s + jnp.exp(xc - m)
            xt = jnp.where(t == c, xc, xt)
        l = jnp.log(s) + (m - xt)
        if write_loss:
            loss_ref[0, rs] = l
        keep = l > _C0
        acc_s = acc_s + jnp.where(keep, l, 0.0)
        acc_c = acc_c + keep.astype(jnp.float32)
    psum = jnp.sum(acc_s)
    pcnt = jnp.sum(acc_c)

    @pl.when(jnp.logical_and(i == 0, j == 0))
    def _():
        sum_ref[0, 0] = 0.0
        cnt_ref[0, 0] = 0.0

    sum_ref[0, 0] += psum
    cnt_ref[0, 0] += pcnt


def _make_stage1(write_loss):
    out_shape = [
        jax.ShapeDtypeStruct((1, 1), jnp.float32),
        jax.ShapeDtypeStruct((1, 1), jnp.float32),
    ]
    out_specs = [
        pl.BlockSpec((1, 1), lambda i, j: (0, 0), memory_space=pltpu.SMEM),
        pl.BlockSpec((1, 1), lambda i, j: (0, 0), memory_space=pltpu.SMEM),
    ]
    if write_loss:
        out_shape = [jax.ShapeDtypeStruct((_B, _H, _W), jnp.float32)] + out_shape
        out_specs = [pl.BlockSpec((1, _BH, _W), lambda i, j: (i, j, 0))] + out_specs
    return pl.pallas_call(
        functools.partial(_ce_body, write_loss),
        grid=(_B, _H // _BH),
        in_specs=[
            pl.BlockSpec((1, _C, _BH, _W), lambda i, j: (i, 0, j, 0)),
            pl.BlockSpec((1, _BH, _W), lambda i, j: (i, j, 0)),
        ],
        out_specs=out_specs,
        out_shape=out_shape,
        compiler_params=pltpu.CompilerParams(
            dimension_semantics=("arbitrary", "arbitrary"),
        ),
    )


_NTILES = 32
_PER_TILE = _N // _NTILES
_CHUNK = 8192
_NBINS = 2048


def _make_hist(level):
    mesh = plsc.VectorSubcoreMesh(core_axis_name="c", subcore_axis_name="s")

    @functools.partial(
        pl.kernel,
        mesh=mesh,
        out_type=(
            jax.ShapeDtypeStruct((_NTILES, _NBINS), jnp.int32),
            jax.ShapeDtypeStruct((_NTILES, _NBINS), jnp.float32),
        ),
        scratch_types=(
            pltpu.VMEM((_CHUNK,), jnp.float32),
            pltpu.VMEM((_NBINS,), jnp.int32),
            pltpu.VMEM((_NBINS,), jnp.float32),
            pltpu.VMEM((16,), jnp.int32),
        ),
        compiler_params=pltpu.CompilerParams(needs_layout_passes=False),
    )
    def hist_k(loss_hbm, pref_hbm, cnt_hbm, sum_hbm, buf, hcnt, hsum, prefv):
        cid = lax.axis_index("c")
        sid = lax.axis_index("s")
        wid = sid * 2 + cid
        base = wid * _PER_TILE
        pltpu.sync_copy(pref_hbm, prefv)
        pv = prefv[...]

        def zero_body(i, carry):
            hcnt[pl.ds(i * 16, 16)] = jnp.zeros((16,), jnp.int32)
            hsum[pl.ds(i * 16, 16)] = jnp.zeros((16,), jnp.float32)
            return carry

        lax.fori_loop(0, _NBINS // 16, zero_body, 0)

        def chunk_body(cix, carry):
            pltpu.sync_copy(loss_hbm.at[pl.ds(base + cix * _CHUNK, _CHUNK)], buf)

            def grp(i, carry2):
                v = buf[pl.ds(i * 16, 16)]
                kb = lax.bitcast_convert_type(v, jnp.int32)
                key = kb ^ (
                    lax.shift_right_arithmetic(kb, 31) & jnp.int32(0x7FFFFFFF)
                )
                if level == 0:
                    mask = None
                    binv = lax.shift_right_arithmetic(key, 21) + 1024
                elif level == 1:
                    mask = lax.shift_right_arithmetic(key, 21) == pv
                    binv = lax.shift_right_logical(key, 10) & jnp.int32(2047)
                else:
                    mask = lax.shift_right_arithmetic(key, 10) == pv
                    binv = key & jnp.int32(1023)
                plsc.addupdate_scatter(
                    hcnt, [binv], jnp.ones((16,), jnp.int32), mask=mask
                )
                plsc.addupdate_scatter(hsum, [binv], v, mask=mask)
                return carry2

            lax.fori_loop(0, _CHUNK // 16, grp, 0)
            return carry

        lax.fori_loop(0, _PER_TILE // _CHUNK, chunk_body, 0)
        pltpu.sync_copy(hcnt, cnt_hbm.at[wid])
        pltpu.sync_copy(hsum, sum_hbm.at[wid])

    return hist_k


@functools.lru_cache(maxsize=None)
def _hist_kernel(level):
    return _make_hist(level)


def _level_reduce(cnt_rows, sum_rows, k):
    cnt = jnp.sum(cnt_rows, axis=0)
    sm = jnp.sum(sum_rows, axis=0)
    cge = jnp.cumsum(cnt[::-1])[::-1]
    sge = jnp.cumsum(sm[::-1])[::-1]
    meets = cge >= (k + 1)
    b = jnp.sum(meets.astype(jnp.int32)) - 1
    above_cnt = cge[b] - cnt[b]
    above_sum = sge[b] - sm[b]
    k_next = k - above_cnt
    return b, k_next, above_cnt, above_sum


def _select_cold(loss_flat):
    z16 = jnp.zeros((16,), jnp.int32)
    c1, s1 = _hist_kernel(0)(loss_flat, z16)
    b1, k1, ac1, as1 = _level_reduce(c1, s1, jnp.int32(_K))
    p1val = b1 - 1024
    c2, s2 = _hist_kernel(1)(loss_flat, jnp.full((16,), 0, jnp.int32) + p1val)
    b2, k2, ac2, as2 = _level_reduce(c2, s2, k1)
    p2val = (p1val << 11) | b2
    c3, s3 = _hist_kernel(2)(loss_flat, jnp.full((16,), 0, jnp.int32) + p2val)
    b3, k3, ac3, as3 = _level_reduce(c3, s3, k2)
    kept_cnt = ac1 + ac2 + ac3
    kept_sum = as1 + as2 + as3
    return kept_sum / jnp.maximum(kept_cnt, 1).astype(jnp.float32)


def kernel(seg_pred, seg_targets):
    psum, pcnt = _make_stage1(False)(seg_pred, seg_targets)
    total_sum = psum[0, 0]
    total_cnt = pcnt[0, 0]

    def _hot(_):
        return total_sum / total_cnt

    def _cold(_):
        outs = _make_stage1(True)(seg_pred, seg_targets)
        loss_flat = outs[0].reshape(-1)
        return _select_cold(loss_flat)

    return lax.cond(total_cnt > float(_K), _hot, _cold, None)


# speedup vs baseline: 61.3496x; 1.1359x over previous
import functools
import math

import jax
import jax.numpy as jnp
from jax import lax
from jax.experimental import pallas as pl
from jax.experimental.pallas import tpu as pltpu
from jax.experimental.pallas import tpu_sc as plsc

_B = 16
_C = 19
_H = 512
_W = 512
_N = _B * _H * _W
_K = 100000
_C0 = float(-math.log(0.7))

_BH = 256


def _ce_body(write_loss, pred_ref, tgt_ref, *out_refs):
    if write_loss:
        loss_ref, sum_ref, cnt_ref = out_refs
    else:
        sum_ref, cnt_ref = out_refs
    i = pl.program_id(0)
    j = pl.program_id(1)
    sub = 8
    acc_s = jnp.zeros((sub, _W), jnp.float32)
    acc_c = jnp.zeros((sub, _W), jnp.float32)
    for r in range(_BH // sub):
        rs = pl.ds(r * sub, sub)
        t = tgt_ref[0, rs]
        m = pred_ref[0, 0, rs]
        for c in range(1, _C):
            m = jnp.maximum(m, pred_ref[0, c, rs])
        s = jnp.zeros_like(m)
        xt = jnp.zeros_like(m)
        for c in range(_C):
            xc = pred_ref[0, c, rs]
            s = s + jnp.exp(xc - m)
            xt = jnp.where(t == c, xc, xt)
        l = jnp.log(s) + (m - xt)
        if write_loss:
            loss_ref[0, rs] = l
        keep = l > _C0
        acc_s = acc_s + jnp.where(keep, l, 0.0)
        acc_c = acc_c + keep.astype(jnp.float32)
    psum = jnp.sum(acc_s)
    pcnt = jnp.sum(acc_c)

    @pl.when(jnp.logical_and(i == 0, j == 0))
    def _():
        sum_ref[0, 0] = 0.0
        cnt_ref[0, 0] = 0.0

    sum_ref[0, 0] += psum
    cnt_ref[0, 0] += pcnt


def _make_stage1(write_loss):
    out_shape = [
        jax.ShapeDtypeStruct((1, 1), jnp.float32),
        jax.ShapeDtypeStruct((1, 1), jnp.float32),
    ]
    out_specs = [
        pl.BlockSpec((1, 1), lambda i, j: (0, 0), memory_space=pltpu.SMEM),
        pl.BlockSpec((1, 1), lambda i, j: (0, 0), memory_space=pltpu.SMEM),
    ]
    if write_loss:
        out_shape = [jax.ShapeDtypeStruct((_B, _H, _W), jnp.float32)] + out_shape
        out_specs = [pl.BlockSpec((1, _BH, _W), lambda i, j: (i, j, 0))] + out_specs
    return pl.pallas_call(
        functools.partial(_ce_body, write_loss),
        grid=(_B, _H // _BH),
        in_specs=[
            pl.BlockSpec((1, _C, _BH, _W), lambda i, j: (i, 0, j, 0)),
            pl.BlockSpec((1, _BH, _W), lambda i, j: (i, j, 0)),
        ],
        out_specs=out_specs,
        out_shape=out_shape,
        compiler_params=pltpu.CompilerParams(
            dimension_semantics=("arbitrary", "arbitrary"),
        ),
    )


_NTILES = 32
_PER_TILE = _N // _NTILES
_CHUNK = 8192
_NBINS = 2048


def _make_hist(level):
    mesh = plsc.VectorSubcoreMesh(core_axis_name="c", subcore_axis_name="s")

    @functools.partial(
        pl.kernel,
        mesh=mesh,
        out_type=(
            jax.ShapeDtypeStruct((_NTILES, _NBINS), jnp.int32),
            jax.ShapeDtypeStruct((_NTILES, _NBINS), jnp.float32),
        ),
        scratch_types=(
            pltpu.VMEM((_CHUNK,), jnp.float32),
            pltpu.VMEM((_NBINS,), jnp.int32),
            pltpu.VMEM((_NBINS,), jnp.float32),
            pltpu.VMEM((16,), jnp.int32),
        ),
        compiler_params=pltpu.CompilerParams(needs_layout_passes=False),
    )
    def hist_k(loss_hbm, pref_hbm, cnt_hbm, sum_hbm, buf, hcnt, hsum, prefv):
        cid = lax.axis_index("c")
        sid = lax.axis_index("s")
        wid = sid * 2 + cid
        base = wid * _PER_TILE
        pltpu.sync_copy(pref_hbm, prefv)
        pv = prefv[...]

        def zero_body(i, carry):
            hcnt[pl.ds(i * 16, 16)] = jnp.zeros((16,), jnp.int32)
            hsum[pl.ds(i * 16, 16)] = jnp.zeros((16,), jnp.float32)
            return carry

        lax.fori_loop(0, _NBINS // 16, zero_body, 0)

        def chunk_body(cix, carry):
            pltpu.sync_copy(loss_hbm.at[pl.ds(base + cix * _CHUNK, _CHUNK)], buf)

            def grp(i, carry2):
                v = buf[pl.ds(i * 16, 16)]
                kb = lax.bitcast_convert_type(v, jnp.int32)
                key = kb ^ (
                    lax.shift_right_arithmetic(kb, 31) & jnp.int32(0x7FFFFFFF)
                )
                if level == 0:
                    mask = None
                    binv = lax.shift_right_arithmetic(key, 21) + 1024
                elif level == 1:
                    mask = lax.shift_right_arithmetic(key, 21) == pv
                    binv = lax.shift_right_logical(key, 10) & jnp.int32(2047)
                else:
                    mask = lax.shift_right_arithmetic(key, 10) == pv
                    binv = key & jnp.int32(1023)
                plsc.addupdate_scatter(
                    hcnt, [binv], jnp.ones((16,), jnp.int32), mask=mask
                )
                plsc.addupdate_scatter(hsum, [binv], v, mask=mask)
                return carry2

            lax.fori_loop(0, _CHUNK // 16, grp, 0)
            return carry

        lax.fori_loop(0, _PER_TILE // _CHUNK, chunk_body, 0)
        pltpu.sync_copy(hcnt, cnt_hbm.at[wid])
        pltpu.sync_copy(hsum, sum_hbm.at[wid])

    return hist_k


@functools.lru_cache(maxsize=None)
def _hist_kernel(level):
    return _make_hist(level)


def _level_reduce(cnt_rows, sum_rows, k):
    cnt = jnp.sum(cnt_rows, axis=0)
    sm = jnp.sum(sum_rows, axis=0)
    cge = jnp.cumsum(cnt[::-1])[::-1]
    sge = jnp.cumsum(sm[::-1])[::-1]
    meets = cge >= (k + 1)
    b = jnp.sum(meets.astype(jnp.int32)) - 1
    above_cnt = cge[b] - cnt[b]
    above_sum = sge[b] - sm[b]
    k_next = k - above_cnt
    return b, k_next, above_cnt, above_sum


def _select_cold(loss_flat):
    z16 = jnp.zeros((16,), jnp.int32)
    c1, s1 = _hist_kernel(0)(loss_flat, z16)
    b1, k1, ac1, as1 = _level_reduce(c1, s1, jnp.int32(_K))
    p1val = b1 - 1024
    c2, s2 = _hist_kernel(1)(loss_flat, jnp.full((16,), 0, jnp.int32) + p1val)
    b2, k2, ac2, as2 = _level_reduce(c2, s2, k1)
    p2val = (p1val << 11) | b2
    c3, s3 = _hist_kernel(2)(loss_flat, jnp.full((16,), 0, jnp.int32) + p2val)
    b3, k3, ac3, as3 = _level_reduce(c3, s3, k2)
    kept_cnt = ac1 + ac2 + ac3
    kept_sum = as1 + as2 + as3
    return kept_sum / jnp.maximum(kept_cnt, 1).astype(jnp.float32)


def kernel(seg_pred, seg_targets):
    psum, pcnt = _make_stage1(False)(seg_pred, seg_targets)
    total_sum = psum[0, 0]
    total_cnt = pcnt[0, 0]

    def _hot(_):
        return total_sum / total_cnt

    def _cold(_):
        outs = _make_stage1(True)(seg_pred, seg_targets)
        loss_flat = outs[0].reshape(-1)
        return _select_cold(loss_flat)

    return lax.cond(total_cnt > float(_K), _hot, _cold, None)


# speedup vs baseline: 65.1384x; 1.0618x over previous
import functools
import math

import jax
import jax.numpy as jnp
from jax import lax
from jax.experimental import pallas as pl
from jax.experimental.pallas import tpu as pltpu
from jax.experimental.pallas import tpu_sc as plsc

_B = 16
_C = 19
_H = 512
_W = 512
_N = _B * _H * _W
_K = 100000
_C0 = float(-math.log(0.7))

_BH = 512


def _ce_body(write_loss, pred_ref, tgt_ref, *out_refs):
    if write_loss:
        loss_ref, sum_ref, cnt_ref = out_refs
    else:
        sum_ref, cnt_ref = out_refs
    i = pl.program_id(0)
    j = pl.program_id(1)
    sub = 8
    acc_s = jnp.zeros((sub, _W), jnp.float32)
    acc_c = jnp.zeros((sub, _W), jnp.float32)
    for r in range(_BH // sub):
        rs = pl.ds(r * sub, sub)
        t = tgt_ref[0, rs]
        m = pred_ref[0, 0, rs]
        for c in range(1, _C):
            m = jnp.maximum(m, pred_ref[0, c, rs])
        s = jnp.zeros_like(m)
        xt = jnp.zeros_like(m)
        for c in range(_C):
            xc = pred_ref[0, c, rs]
            s = s + jnp.exp(xc - m)
            xt = jnp.where(t == c, xc, xt)
        l = jnp.log(s) + (m - xt)
        if write_loss:
            loss_ref[0, rs] = l
        keep = l > _C0
        acc_s = acc_s + jnp.where(keep, l, 0.0)
        acc_c = acc_c + keep.astype(jnp.float32)
    psum = jnp.sum(acc_s)
    pcnt = jnp.sum(acc_c)

    @pl.when(jnp.logical_and(i == 0, j == 0))
    def _():
        sum_ref[0, 0] = 0.0
        cnt_ref[0, 0] = 0.0

    sum_ref[0, 0] += psum
    cnt_ref[0, 0] += pcnt


def _make_stage1(write_loss):
    out_shape = [
        jax.ShapeDtypeStruct((1, 1), jnp.float32),
        jax.ShapeDtypeStruct((1, 1), jnp.float32),
    ]
    out_specs = [
        pl.BlockSpec((1, 1), lambda i, j: (0, 0), memory_space=pltpu.SMEM),
        pl.BlockSpec((1, 1), lambda i, j: (0, 0), memory_space=pltpu.SMEM),
    ]
    if write_loss:
        out_shape = [jax.ShapeDtypeStruct((_B, _H, _W), jnp.float32)] + out_shape
        out_specs = [pl.BlockSpec((1, _BH, _W), lambda i, j: (i, j, 0))] + out_specs
    return pl.pallas_call(
        functools.partial(_ce_body, write_loss),
        grid=(_B, _H // _BH),
        in_specs=[
            pl.BlockSpec((1, _C, _BH, _W), lambda i, j: (i, 0, j, 0)),
            pl.BlockSpec((1, _BH, _W), lambda i, j: (i, j, 0)),
        ],
        out_specs=out_specs,
        out_shape=out_shape,
        compiler_params=pltpu.CompilerParams(
            dimension_semantics=("arbitrary", "arbitrary"),
        ),
    )


_NTILES = 32
_PER_TILE = _N // _NTILES
_CHUNK = 8192
_NBINS = 2048


def _make_hist(level):
    mesh = plsc.VectorSubcoreMesh(core_axis_name="c", subcore_axis_name="s")

    @functools.partial(
        pl.kernel,
        mesh=mesh,
        out_type=(
            jax.ShapeDtypeStruct((_NTILES, _NBINS), jnp.int32),
            jax.ShapeDtypeStruct((_NTILES, _NBINS), jnp.float32),
        ),
        scratch_types=(
            pltpu.VMEM((_CHUNK,), jnp.float32),
            pltpu.VMEM((_NBINS,), jnp.int32),
            pltpu.VMEM((_NBINS,), jnp.float32),
            pltpu.VMEM((16,), jnp.int32),
        ),
        compiler_params=pltpu.CompilerParams(needs_layout_passes=False),
    )
    def hist_k(loss_hbm, pref_hbm, cnt_hbm, sum_hbm, buf, hcnt, hsum, prefv):
        cid = lax.axis_index("c")
        sid = lax.axis_index("s")
        wid = sid * 2 + cid
        base = wid * _PER_TILE
        pltpu.sync_copy(pref_hbm, prefv)
        pv = prefv[...]

        def zero_body(i, carry):
            hcnt[pl.ds(i * 16, 16)] = jnp.zeros((16,), jnp.int32)
            hsum[pl.ds(i * 16, 16)] = jnp.zeros((16,), jnp.float32)
            return carry

        lax.fori_loop(0, _NBINS // 16, zero_body, 0)

        def chunk_body(cix, carry):
            pltpu.sync_copy(loss_hbm.at[pl.ds(base + cix * _CHUNK, _CHUNK)], buf)

            def grp(i, carry2):
                v = buf[pl.ds(i * 16, 16)]
                kb = lax.bitcast_convert_type(v, jnp.int32)
                key = kb ^ (
                    lax.shift_right_arithmetic(kb, 31) & jnp.int32(0x7FFFFFFF)
                )
                if level == 0:
                    mask = None
                    binv = lax.shift_right_arithmetic(key, 21) + 1024
                elif level == 1:
                    mask = lax.shift_right_arithmetic(key, 21) == pv
                    binv = lax.shift_right_logical(key, 10) & jnp.int32(2047)
                else:
                    mask = lax.shift_right_arithmetic(key, 10) == pv
                    binv = key & jnp.int32(1023)
                plsc.addupdate_scatter(
                    hcnt, [binv], jnp.ones((16,), jnp.int32), mask=mask
                )
                plsc.addupdate_scatter(hsum, [binv], v, mask=mask)
                return carry2

            lax.fori_loop(0, _CHUNK // 16, grp, 0)
            return carry

        lax.fori_loop(0, _PER_TILE // _CHUNK, chunk_body, 0)
        pltpu.sync_copy(hcnt, cnt_hbm.at[wid])
        pltpu.sync_copy(hsum, sum_hbm.at[wid])

    return hist_k


@functools.lru_cache(maxsize=None)
def _hist_kernel(level):
    return _make_hist(level)


def _level_reduce(cnt_rows, sum_rows, k):
    cnt = jnp.sum(cnt_rows, axis=0)
    sm = jnp.sum(sum_rows, axis=0)
    cge = jnp.cumsum(cnt[::-1])[::-1]
    sge = jnp.cumsum(sm[::-1])[::-1]
    meets = cge >= (k + 1)
    b = jnp.sum(meets.astype(jnp.int32)) - 1
    above_cnt = cge[b] - cnt[b]
    above_sum = sge[b] - sm[b]
    k_next = k - above_cnt
    return b, k_next, above_cnt, above_sum


def _select_cold(loss_flat):
    z16 = jnp.zeros((16,), jnp.int32)
    c1, s1 = _hist_kernel(0)(loss_flat, z16)
    b1, k1, ac1, as1 = _level_reduce(c1, s1, jnp.int32(_K))
    p1val = b1 - 1024
    c2, s2 = _hist_kernel(1)(loss_flat, jnp.full((16,), 0, jnp.int32) + p1val)
    b2, k2, ac2, as2 = _level_reduce(c2, s2, k1)
    p2val = (p1val << 11) | b2
    c3, s3 = _hist_kernel(2)(loss_flat, jnp.full((16,), 0, jnp.int32) + p2val)
    b3, k3, ac3, as3 = _level_reduce(c3, s3, k2)
    kept_cnt = ac1 + ac2 + ac3
    kept_sum = as1 + as2 + as3
    return kept_sum / jnp.maximum(kept_cnt, 1).astype(jnp.float32)


def kernel(seg_pred, seg_targets):
    psum, pcnt = _make_stage1(False)(seg_pred, seg_targets)
    total_sum = psum[0, 0]
    total_cnt = pcnt[0, 0]

    def _hot(_):
        return total_sum / total_cnt

    def _cold(_):
        outs = _make_stage1(True)(seg_pred, seg_targets)
        loss_flat = outs[0].reshape(-1)
        return _select_cold(loss_flat)

    return lax.cond(total_cnt > float(_K), _hot, _cold, None)


# speedup vs baseline: 65.1845x; 1.0007x over previous
import functools
import math

import jax
import jax.numpy as jnp
from jax import lax
from jax.experimental import pallas as pl
from jax.experimental.pallas import tpu as pltpu
from jax.experimental.pallas import tpu_sc as plsc

_B = 16
_C = 19
_H = 512
_W = 512
_N = _B * _H * _W
_K = 100000
_C0 = float(-math.log(0.7))

_BH = 512


_NQ = 4


def _ce_body(write_loss, *refs):
    pred_refs = refs[:_NQ]
    tgt_ref = refs[_NQ]
    if write_loss:
        loss_ref, sum_ref, cnt_ref = refs[_NQ + 1 :]
    else:
        sum_ref, cnt_ref = refs[_NQ + 1 :]
    i = pl.program_id(0)
    qh = _BH // _NQ
    sub = 8
    acc_s = jnp.zeros((sub, _W), jnp.float32)
    acc_c = jnp.zeros((sub, _W), jnp.float32)
    for q in range(_NQ):
        pred_ref = pred_refs[q]
        for r in range(qh // sub):
            rs = pl.ds(r * sub, sub)
            grs = pl.ds(q * qh + r * sub, sub)
            t = tgt_ref[0, grs]
            m = pred_ref[0, 0, rs]
            for c in range(1, _C):
                m = jnp.maximum(m, pred_ref[0, c, rs])
            s = jnp.zeros_like(m)
            xt = jnp.zeros_like(m)
            for c in range(_C):
                xc = pred_ref[0, c, rs]
                s = s + jnp.exp(xc - m)
                xt = jnp.where(t == c, xc, xt)
            l = jnp.log(s) + (m - xt)
            if write_loss:
                loss_ref[0, grs] = l
            keep = l > _C0
            acc_s = acc_s + jnp.where(keep, l, 0.0)
            acc_c = acc_c + keep.astype(jnp.float32)
    psum = jnp.sum(acc_s)
    pcnt = jnp.sum(acc_c)

    @pl.when(i == 0)
    def _():
        sum_ref[0, 0] = 0.0
        cnt_ref[0, 0] = 0.0

    sum_ref[0, 0] += psum
    cnt_ref[0, 0] += pcnt


def _make_stage1(write_loss):
    qh = _BH // _NQ
    out_shape = [
        jax.ShapeDtypeStruct((1, 1), jnp.float32),
        jax.ShapeDtypeStruct((1, 1), jnp.float32),
    ]
    out_specs = [
        pl.BlockSpec((1, 1), lambda i: (0, 0), memory_space=pltpu.SMEM),
        pl.BlockSpec((1, 1), lambda i: (0, 0), memory_space=pltpu.SMEM),
    ]
    if write_loss:
        out_shape = [jax.ShapeDtypeStruct((_B, _H, _W), jnp.float32)] + out_shape
        out_specs = [pl.BlockSpec((1, _BH, _W), lambda i: (i, 0, 0))] + out_specs

    def _mk_pred_spec(q):
        return pl.BlockSpec((1, _C, qh, _W), lambda i: (i, 0, q, 0))

    return pl.pallas_call(
        functools.partial(_ce_body, write_loss),
        grid=(_B,),
        in_specs=[_mk_pred_spec(q) for q in range(_NQ)]
        + [pl.BlockSpec((1, _BH, _W), lambda i: (i, 0, 0))],
        out_specs=out_specs,
        out_shape=out_shape,
        compiler_params=pltpu.CompilerParams(
            dimension_semantics=("arbitrary",),
        ),
    )


_NTILES = 32
_PER_TILE = _N // _NTILES
_CHUNK = 8192
_NBINS = 2048


def _make_hist(level):
    mesh = plsc.VectorSubcoreMesh(core_axis_name="c", subcore_axis_name="s")

    @functools.partial(
        pl.kernel,
        mesh=mesh,
        out_type=(
            jax.ShapeDtypeStruct((_NTILES, _NBINS), jnp.int32),
            jax.ShapeDtypeStruct((_NTILES, _NBINS), jnp.float32),
        ),
        scratch_types=(
            pltpu.VMEM((_CHUNK,), jnp.float32),
            pltpu.VMEM((_NBINS,), jnp.int32),
            pltpu.VMEM((_NBINS,), jnp.float32),
            pltpu.VMEM((16,), jnp.int32),
        ),
        compiler_params=pltpu.CompilerParams(needs_layout_passes=False),
    )
    def hist_k(loss_hbm, pref_hbm, cnt_hbm, sum_hbm, buf, hcnt, hsum, prefv):
        cid = lax.axis_index("c")
        sid = lax.axis_index("s")
        wid = sid * 2 + cid
        base = wid * _PER_TILE
        pltpu.sync_copy(pref_hbm, prefv)
        pv = prefv[...]

        def zero_body(i, carry):
            hcnt[pl.ds(i * 16, 16)] = jnp.zeros((16,), jnp.int32)
            hsum[pl.ds(i * 16, 16)] = jnp.zeros((16,), jnp.float32)
            return carry

        lax.fori_loop(0, _NBINS // 16, zero_body, 0)

        def chunk_body(cix, carry):
            pltpu.sync_copy(loss_hbm.at[pl.ds(base + cix * _CHUNK, _CHUNK)], buf)

            def grp(i, carry2):
                v = buf[pl.ds(i * 16, 16)]
                kb = lax.bitcast_convert_type(v, jnp.int32)
                key = kb ^ (
                    lax.shift_right_arithmetic(kb, 31) & jnp.int32(0x7FFFFFFF)
                )
                if level == 0:
                    mask = None
                    binv = lax.shift_right_arithmetic(key, 21) + 1024
                elif level == 1:
                    mask = lax.shift_right_arithmetic(key, 21) == pv
                    binv = lax.shift_right_logical(key, 10) & jnp.int32(2047)
                else:
                    mask = lax.shift_right_arithmetic(key, 10) == pv
                    binv = key & jnp.int32(1023)
                plsc.addupdate_scatter(
                    hcnt, [binv], jnp.ones((16,), jnp.int32), mask=mask
                )
                plsc.addupdate_scatter(hsum, [binv], v, mask=mask)
                return carry2

            lax.fori_loop(0, _CHUNK // 16, grp, 0)
            return carry

        lax.fori_loop(0, _PER_TILE // _CHUNK, chunk_body, 0)
        pltpu.sync_copy(hcnt, cnt_hbm.at[wid])
        pltpu.sync_copy(hsum, sum_hbm.at[wid])

    return hist_k


@functools.lru_cache(maxsize=None)
def _hist_kernel(level):
    return _make_hist(level)


def _level_reduce(cnt_rows, sum_rows, k):
    cnt = jnp.sum(cnt_rows, axis=0)
    sm = jnp.sum(sum_rows, axis=0)
    cge = jnp.cumsum(cnt[::-1])[::-1]
    sge = jnp.cumsum(sm[::-1])[::-1]
    meets = cge >= (k + 1)
    b = jnp.sum(meets.astype(jnp.int32)) - 1
    above_cnt = cge[b] - cnt[b]
    above_sum = sge[b] - sm[b]
    k_next = k - above_cnt
    return b, k_next, above_cnt, above_sum


def _select_cold(loss_flat):
    z16 = jnp.zeros((16,), jnp.int32)
    c1, s1 = _hist_kernel(0)(loss_flat, z16)
    b1, k1, ac1, as1 = _level_reduce(c1, s1, jnp.int32(_K))
    p1val = b1 - 1024
    c2, s2 = _hist_kernel(1)(loss_flat, jnp.full((16,), 0, jnp.int32) + p1val)
    b2, k2, ac2, as2 = _level_reduce(c2, s2, k1)
    p2val = (p1val << 11) | b2
    c3, s3 = _hist_kernel(2)(loss_flat, jnp.full((16,), 0, jnp.int32) + p2val)
    b3, k3, ac3, as3 = _level_reduce(c3, s3, k2)
    kept_cnt = ac1 + ac2 + ac3
    kept_sum = as1 + as2 + as3
    return kept_sum / jnp.maximum(kept_cnt, 1).astype(jnp.float32)


def kernel(seg_pred, seg_targets):
    psum, pcnt = _make_stage1(False)(*([seg_pred] * _NQ), seg_targets)
    total_sum = psum[0, 0]
    total_cnt = pcnt[0, 0]

    def _hot(_):
        return total_sum / total_cnt

    def _cold(_):
        outs = _make_stage1(True)(*([seg_pred] * _NQ), seg_targets)
        loss_flat = outs[0].reshape(-1)
        return _select_cold(loss_flat)

    return lax.cond(total_cnt > float(_K), _hot, _cold, None)


# speedup vs baseline: 65.2514x; 1.0010x over previous
import functools
import math

import jax
import jax.numpy as jnp
from jax import lax
from jax.experimental import pallas as pl
from jax.experimental.pallas import tpu as pltpu
from jax.experimental.pallas import tpu_sc as plsc

_B = 16
_C = 19
_H = 512
_W = 512
_N = _B * _H * _W
_K = 100000
_C0 = float(-math.log(0.7))

_BH = 512


def _ce_body(write_loss, pred_ref, tgt_ref, *out_refs):
    if write_loss:
        loss_ref, sum_ref, cnt_ref = out_refs
    else:
        sum_ref, cnt_ref = out_refs
    i = pl.program_id(0)
    j = pl.program_id(1)
    sub = 8
    acc_s = jnp.zeros((sub, _W), jnp.float32)
    acc_c = jnp.zeros((sub, _W), jnp.float32)
    for r in range(_BH // sub):
        rs = pl.ds(r * sub, sub)
        t = tgt_ref[0, rs]
        m = pred_ref[0, 0, rs]
        for c in range(1, _C):
            m = jnp.maximum(m, pred_ref[0, c, rs])
        s = jnp.zeros_like(m)
        xt = jnp.zeros_like(m)
        for c in range(_C):
            xc = pred_ref[0, c, rs]
            s = s + jnp.exp(xc - m)
            xt = jnp.where(t == c, xc, xt)
        l = jnp.log(s) + (m - xt)
        if write_loss:
            loss_ref[0, rs] = l
        keep = l > _C0
        acc_s = acc_s + jnp.where(keep, l, 0.0)
        acc_c = acc_c + keep.astype(jnp.float32)
    psum = jnp.sum(acc_s)
    pcnt = jnp.sum(acc_c)

    @pl.when(jnp.logical_and(i == 0, j == 0))
    def _():
        sum_ref[0, 0] = 0.0
        cnt_ref[0, 0] = 0.0

    sum_ref[0, 0] += psum
    cnt_ref[0, 0] += pcnt


def _make_stage1(write_loss):
    out_shape = [
        jax.ShapeDtypeStruct((1, 1), jnp.float32),
        jax.ShapeDtypeStruct((1, 1), jnp.float32),
    ]
    out_specs = [
        pl.BlockSpec((1, 1), lambda i, j: (0, 0), memory_space=pltpu.SMEM),
        pl.BlockSpec((1, 1), lambda i, j: (0, 0), memory_space=pltpu.SMEM),
    ]
    if write_loss:
        out_shape = [jax.ShapeDtypeStruct((_B, _H, _W), jnp.float32)] + out_shape
        out_specs = [pl.BlockSpec((1, _BH, _W), lambda i, j: (i, j, 0))] + out_specs
    return pl.pallas_call(
        functools.partial(_ce_body, write_loss),
        grid=(_B, _H // _BH),
        in_specs=[
            pl.BlockSpec((1, _C, _BH, _W), lambda i, j: (i, 0, j, 0)),
            pl.BlockSpec((1, _BH, _W), lambda i, j: (i, j, 0)),
        ],
        out_specs=out_specs,
        out_shape=out_shape,
        compiler_params=pltpu.CompilerParams(
            dimension_semantics=("arbitrary", "arbitrary"),
        ),
    )


_NTILES = 32
_PER_TILE = _N // _NTILES
_CHUNK = 8192
_NBINS = 2048


def _make_hist(level):
    mesh = plsc.VectorSubcoreMesh(core_axis_name="c", subcore_axis_name="s")

    @functools.partial(
        pl.kernel,
        mesh=mesh,
        out_type=(
            jax.ShapeDtypeStruct((_NTILES, _NBINS), jnp.int32),
            jax.ShapeDtypeStruct((_NTILES, _NBINS), jnp.float32),
        ),
        scratch_types=(
            pltpu.VMEM((_CHUNK,), jnp.float32),
            pltpu.VMEM((_NBINS,), jnp.int32),
            pltpu.VMEM((_NBINS,), jnp.float32),
            pltpu.VMEM((16,), jnp.int32),
        ),
        compiler_params=pltpu.CompilerParams(needs_layout_passes=False),
    )
    def hist_k(loss_hbm, pref_hbm, cnt_hbm, sum_hbm, buf, hcnt, hsum, prefv):
        cid = lax.axis_index("c")
        sid = lax.axis_index("s")
        wid = sid * 2 + cid
        base = wid * _PER_TILE
        pltpu.sync_copy(pref_hbm, prefv)
        pv = prefv[...]

        def zero_body(i, carry):
            hcnt[pl.ds(i * 16, 16)] = jnp.zeros((16,), jnp.int32)
            hsum[pl.ds(i * 16, 16)] = jnp.zeros((16,), jnp.float32)
            return carry

        lax.fori_loop(0, _NBINS // 16, zero_body, 0)

        def chunk_body(cix, carry):
            pltpu.sync_copy(loss_hbm.at[pl.ds(base + cix * _CHUNK, _CHUNK)], buf)

            def grp(i, carry2):
                v = buf[pl.ds(i * 16, 16)]
                kb = lax.bitcast_convert_type(v, jnp.int32)
                key = kb ^ (
                    lax.shift_right_arithmetic(kb, 31) & jnp.int32(0x7FFFFFFF)
                )
                if level == 0:
                    mask = None
                    binv = lax.shift_right_arithmetic(key, 21) + 1024
                elif level == 1:
                    mask = lax.shift_right_arithmetic(key, 21) == pv
                    binv = lax.shift_right_logical(key, 10) & jnp.int32(2047)
                else:
                    mask = lax.shift_right_arithmetic(key, 10) == pv
                    binv = key & jnp.int32(1023)
                plsc.addupdate_scatter(
                    hcnt, [binv], jnp.ones((16,), jnp.int32), mask=mask
                )
                plsc.addupdate_scatter(hsum, [binv], v, mask=mask)
                return carry2

            lax.fori_loop(0, _CHUNK // 16, grp, 0)
            return carry

        lax.fori_loop(0, _PER_TILE // _CHUNK, chunk_body, 0)
        pltpu.sync_copy(hcnt, cnt_hbm.at[wid])
        pltpu.sync_copy(hsum, sum_hbm.at[wid])

    return hist_k


@functools.lru_cache(maxsize=None)
def _hist_kernel(level):
    return _make_hist(level)


def _level_reduce(cnt_rows, sum_rows, k):
    cnt = jnp.sum(cnt_rows, axis=0)
    sm = jnp.sum(sum_rows, axis=0)
    cge = jnp.cumsum(cnt[::-1])[::-1]
    sge = jnp.cumsum(sm[::-1])[::-1]
    meets = cge >= (k + 1)
    b = jnp.sum(meets.astype(jnp.int32)) - 1
    above_cnt = cge[b] - cnt[b]
    above_sum = sge[b] - sm[b]
    k_next = k - above_cnt
    return b, k_next, above_cnt, above_sum


def _select_cold(loss_flat):
    z16 = jnp.zeros((16,), jnp.int32)
    c1, s1 = _hist_kernel(0)(loss_flat, z16)
    b1, k1, ac1, as1 = _level_reduce(c1, s1, jnp.int32(_K))
    p1val = b1 - 1024
    c2, s2 = _hist_kernel(1)(loss_flat, jnp.full((16,), 0, jnp.int32) + p1val)
    b2, k2, ac2, as2 = _level_reduce(c2, s2, k1)
    p2val = (p1val << 11) | b2
    c3, s3 = _hist_kernel(2)(loss_flat, jnp.full((16,), 0, jnp.int32) + p2val)
    b3, k3, ac3, as3 = _level_reduce(c3, s3, k2)
    kept_cnt = ac1 + ac2 + ac3
    kept_sum = as1 + as2 + as3
    return kept_sum / jnp.maximum(kept_cnt, 1).astype(jnp.float32)


def kernel(seg_pred, seg_targets):
    psum, pcnt = _make_stage1(False)(seg_pred, seg_targets)
    total_sum = psum[0, 0]
    total_cnt = pcnt[0, 0]

    def _hot(_):
        return total_sum / total_cnt

    def _cold(_):
        outs = _make_stage1(True)(seg_pred, seg_targets)
        loss_flat = outs[0].reshape(-1)
        return _select_cold(loss_flat)

    return lax.cond(total_cnt > float(_K), _hot, _cold, None)
